# Initial kernel scaffold; baseline (speedup 1.0000x reference)
import functools

import jax
import jax.numpy as jnp
from jax import lax
from jax.experimental import pallas as pl
from jax.experimental.pallas import tpu as pltpu
from jax.experimental.pallas import tpu_sc as plsc

N = 10000
E = 320000
D = 128

NC = 2
NS = 16
NW = NC * NS
C = 128

EPW = -(-E // (NW * C)) * C
E_PAD = EPW * NW
G = EPW // C
ROWS_PER_TILE = N // NS


def _sc_body(x_hbm, row_hbm, col_hbm, w_hbm, out_hbm,
             agg_sh, row_v, col_v, w_v, rows_v, sem):
    c = lax.axis_index("c")
    s = lax.axis_index("s")
    wid = s * NC + c

    zeros16 = jnp.zeros((16,), jnp.float32)

    def zero_row(i, carry):
        for j in range(D // 16):
            rows_v[i, pl.ds(j * 16, 16)] = zeros16
        return carry

    lax.fori_loop(0, C, zero_row, 0)

    t0 = s * ROWS_PER_TILE
    n_full = ROWS_PER_TILE // C
    rem = ROWS_PER_TILE - n_full * C
    for k in range(n_full):
        pltpu.sync_copy(rows_v, agg_sh.at[pl.ds(t0 + k * C, C)])
    if rem:
        pltpu.sync_copy(rows_v.at[pl.ds(0, rem)],
                        agg_sh.at[pl.ds(t0 + n_full * C, rem)])

    plsc.subcore_barrier()

    def chunk(g, carry):
        base = wid * EPW + g * C
        pltpu.sync_copy(row_hbm.at[pl.ds(base, C)], row_v)
        pltpu.sync_copy(w_hbm.at[pl.ds(base, C)], w_v)
        pltpu.sync_copy(col_hbm.at[pl.ds(base, C)], col_v)
        pltpu.async_copy(x_hbm.at[row_v], rows_v, sem).wait()

        def scale(e, inner):
            w = w_v[e]
            for j in range(D // 16):
                sl = pl.ds(j * 16, 16)
                rows_v[e, sl] = rows_v[e, sl] * w
            return inner

        lax.fori_loop(0, C, scale, 0)
        pltpu.sync_copy(rows_v, agg_sh.at[col_v], add=True)
        return carry

    lax.fori_loop(0, G, chunk, 0)

    plsc.subcore_barrier()

    out_row = c * N + t0
    pltpu.sync_copy(agg_sh.at[pl.ds(t0, ROWS_PER_TILE)],
                    out_hbm.at[pl.ds(out_row, ROWS_PER_TILE)])


_sc_agg = pl.kernel(
    _sc_body,
    out_type=jax.ShapeDtypeStruct((NC * N, D), jnp.float32),
    mesh=plsc.VectorSubcoreMesh(core_axis_name="c", subcore_axis_name="s",
                                num_cores=NC, num_subcores=NS),
    scratch_types=[
        pltpu.VMEM_SHARED((N, D), jnp.float32),
        pltpu.VMEM((C,), jnp.int32),
        pltpu.VMEM((C,), jnp.int32),
        pltpu.VMEM((C,), jnp.float32),
        pltpu.VMEM((C, D), jnp.float32),
        pltpu.SemaphoreType.DMA,
    ],
)


BN = 1000


def _tc_body(x_ref, p0_ref, p1_ref, w1t_ref, w2t_ref, b_ref, out_ref):
    a = p0_ref[...] + p1_ref[...]
    xb = x_ref[...]
    su = a + xb
    mu = a * xb
    h = (jnp.dot(su, w1t_ref[...], preferred_element_type=jnp.float32)
         + jnp.dot(mu, w2t_ref[...], preferred_element_type=jnp.float32)
         + b_ref[...])
    out_ref[...] = jnp.where(h >= 0, h, 0.2 * h)


def _tc_dense(x, parts, w1t, w2t, b):
    nb = N // BN
    return pl.pallas_call(
        _tc_body,
        grid=(nb,),
        in_specs=[
            pl.BlockSpec((BN, D), lambda i: (i, 0)),
            pl.BlockSpec((BN, D), lambda i: (i, 0)),
            pl.BlockSpec((BN, D), lambda i: (i + nb, 0)),
            pl.BlockSpec((D, D), lambda i: (0, 0)),
            pl.BlockSpec((D, D), lambda i: (0, 0)),
            pl.BlockSpec((1, D), lambda i: (0, 0)),
        ],
        out_specs=pl.BlockSpec((BN, D), lambda i: (i, 0)),
        out_shape=jax.ShapeDtypeStruct((N, D), jnp.float32),
    )(x, parts, parts, w1t, w2t, b)


@jax.jit
def kernel(x, edge_index, edge_weight, W1, b1, W2, b2):
    row = edge_index[0].astype(jnp.int32)
    col = edge_index[1].astype(jnp.int32)
    pad = E_PAD - E
    row = jnp.concatenate([row, jnp.zeros((pad,), jnp.int32)])
    col = jnp.concatenate([col, jnp.zeros((pad,), jnp.int32)])
    w = jnp.concatenate([edge_weight, jnp.zeros((pad,), jnp.float32)])

    parts = _sc_agg(x, row, col, w)

    w1t = W1.T
    w2t = W2.T
    b = (b1 + b2).reshape(1, D)
    return _tc_dense(x, parts, w1t, w2t, b)


# speedup vs baseline: 3.9645x; 3.9645x over previous
import functools

import jax
import jax.numpy as jnp
from jax import lax
from jax.experimental import pallas as pl
from jax.experimental.pallas import tpu as pltpu
from jax.experimental.pallas import tpu_sc as plsc

N = 10000
E = 320000
D = 128

NC = 2
NS = 16
NW = NC * NS
C = 128

EPW = -(-E // (NW * C)) * C
E_PAD = EPW * NW
G = EPW // C
ROWS_PER_TILE = -(-N // (NS * 8)) * 8
N_SP = ROWS_PER_TILE * NS


def _sc_body(x_hbm, row_hbm, col_hbm, w_hbm, out_hbm,
             agg_sh, row_v, col_v, w_v, rows_v, sem):
    c = lax.axis_index("c")
    s = lax.axis_index("s")
    wid = s * NC + c

    zeros16 = jnp.zeros((16,), jnp.float32)

    def zero_row(i, carry):
        for j in range(D // 16):
            rows_v[i, pl.ds(j * 16, 16)] = zeros16
        return carry

    lax.fori_loop(0, C, zero_row, 0)

    t0 = s * ROWS_PER_TILE
    n_full = ROWS_PER_TILE // C
    rem = ROWS_PER_TILE - n_full * C
    for k in range(n_full):
        pltpu.sync_copy(rows_v, agg_sh.at[pl.ds(t0 + k * C, C)])
    if rem:
        pltpu.sync_copy(rows_v.at[pl.ds(0, rem)],
                        agg_sh.at[pl.ds(t0 + n_full * C, rem)])

    plsc.subcore_barrier()

    def chunk(g, carry):
        base = wid * EPW + g * C
        pltpu.sync_copy(row_hbm.at[pl.ds(base, C)], row_v)
        pltpu.sync_copy(w_hbm.at[pl.ds(base, C)], w_v)
        pltpu.sync_copy(col_hbm.at[pl.ds(base, C)], col_v)
        pltpu.async_copy(x_hbm.at[row_v], rows_v, sem).wait()

        def scale(grp, inner):
            wv = w_v[pl.ds(grp * 16, 16)]
            base16 = grp * 16
            for i in range(16):
                w = wv[i]
                for j in range(D // 16):
                    sl = pl.ds(j * 16, 16)
                    rows_v[base16 + i, sl] = rows_v[base16 + i, sl] * w
            return inner

        lax.fori_loop(0, C // 16, scale, 0)
        pltpu.sync_copy(rows_v, agg_sh.at[col_v], add=True)
        return carry

    lax.fori_loop(0, G, chunk, 0)

    plsc.subcore_barrier()

    out_row = c * N + t0
    n_live = N - (NS - 1) * ROWS_PER_TILE

    @pl.when(s < NS - 1)
    def _():
        pltpu.sync_copy(agg_sh.at[pl.ds(t0, ROWS_PER_TILE)],
                        out_hbm.at[pl.ds(out_row, ROWS_PER_TILE)])

    @pl.when(s == NS - 1)
    def _():
        pltpu.sync_copy(agg_sh.at[pl.ds(t0, n_live)],
                        out_hbm.at[pl.ds(out_row, n_live)])


_sc_agg = pl.kernel(
    _sc_body,
    out_type=jax.ShapeDtypeStruct((NC * N, D), jnp.float32),
    mesh=plsc.VectorSubcoreMesh(core_axis_name="c", subcore_axis_name="s",
                                num_cores=NC, num_subcores=NS),
    scratch_types=[
        pltpu.VMEM_SHARED((N_SP, D), jnp.float32),
        pltpu.VMEM((C,), jnp.int32),
        pltpu.VMEM((C,), jnp.int32),
        pltpu.VMEM((C,), jnp.float32),
        pltpu.VMEM((C, D), jnp.float32),
        pltpu.SemaphoreType.DMA,
    ],
)


BN = 1000


def _tc_body(x_ref, p0_ref, p1_ref, w1t_ref, w2t_ref, b_ref, out_ref):
    a = p0_ref[...] + p1_ref[...]
    xb = x_ref[...]
    su = a + xb
    mu = a * xb
    h = (jnp.dot(su, w1t_ref[...], preferred_element_type=jnp.float32)
         + jnp.dot(mu, w2t_ref[...], preferred_element_type=jnp.float32)
         + b_ref[...])
    out_ref[...] = jnp.where(h >= 0, h, 0.2 * h)


def _tc_dense(x, parts, w1t, w2t, b):
    nb = N // BN
    return pl.pallas_call(
        _tc_body,
        grid=(nb,),
        in_specs=[
            pl.BlockSpec((BN, D), lambda i: (i, 0)),
            pl.BlockSpec((BN, D), lambda i: (i, 0)),
            pl.BlockSpec((BN, D), lambda i: (i + nb, 0)),
            pl.BlockSpec((D, D), lambda i: (0, 0)),
            pl.BlockSpec((D, D), lambda i: (0, 0)),
            pl.BlockSpec((1, D), lambda i: (0, 0)),
        ],
        out_specs=pl.BlockSpec((BN, D), lambda i: (i, 0)),
        out_shape=jax.ShapeDtypeStruct((N, D), jnp.float32),
    )(x, parts, parts, w1t, w2t, b)


@jax.jit
def kernel(x, edge_index, edge_weight, W1, b1, W2, b2):
    row = edge_index[0].astype(jnp.int32)
    col = edge_index[1].astype(jnp.int32)
    pad = E_PAD - E
    row = jnp.concatenate([row, jnp.zeros((pad,), jnp.int32)])
    col = jnp.concatenate([col, jnp.zeros((pad,), jnp.int32)])
    w = jnp.concatenate([edge_weight, jnp.zeros((pad,), jnp.float32)])

    parts = _sc_agg(x, row, col, w)

    w1t = W1.T
    w2t = W2.T
    b = (b1 + b2).reshape(1, D)
    return _tc_dense(x, parts, w1t, w2t, b)


# speedup vs baseline: 3.9960x; 1.0079x over previous
import functools

import jax
import jax.numpy as jnp
from jax import lax
from jax.experimental import pallas as pl
from jax.experimental.pallas import tpu as pltpu
from jax.experimental.pallas import tpu_sc as plsc

N = 10000
E = 320000
D = 128

NC = 2
NS = 16
NW = NC * NS
C = 128

EPW = -(-E // (NW * 2 * C)) * 2 * C
E_PAD = EPW * NW
G = EPW // C
ROWS_PER_TILE = -(-N // (NS * 8)) * 8
N_SP = ROWS_PER_TILE * NS


def _sc_body(x_hbm, row_hbm, col_hbm, w_hbm, out_hbm,
             agg_sh, row_all, w_b, col_b, rows_b, gsem, csem, wsem, ssem):
    c = lax.axis_index("c")
    s = lax.axis_index("s")
    wid = s * NC + c
    ebase = wid * EPW

    zeros16 = jnp.zeros((16,), jnp.float32)

    def zero_row(i, carry):
        for j in range(D // 16):
            rows_b[0][i, pl.ds(j * 16, 16)] = zeros16
        return carry

    lax.fori_loop(0, C, zero_row, 0)

    t0 = s * ROWS_PER_TILE
    n_full = ROWS_PER_TILE // C
    rem = ROWS_PER_TILE - n_full * C
    for k in range(n_full):
        pltpu.sync_copy(rows_b[0], agg_sh.at[pl.ds(t0 + k * C, C)])
    if rem:
        pltpu.sync_copy(rows_b[0].at[pl.ds(0, rem)],
                        agg_sh.at[pl.ds(t0 + n_full * C, rem)])

    pltpu.sync_copy(row_hbm.at[pl.ds(ebase, EPW)], row_all)

    plsc.subcore_barrier()

    def issue(g, b):
        pltpu.async_copy(col_hbm.at[pl.ds(ebase + g * C, C)], col_b[b],
                         csem[b])
        pltpu.async_copy(w_hbm.at[pl.ds(ebase + g * C, C)], w_b[b],
                         wsem[b])
        pltpu.async_copy(x_hbm.at[row_all.at[pl.ds(g * C, C)]], rows_b[b],
                         gsem[b])

    def wait_scatter(b):
        pltpu.make_async_copy(rows_b[b], agg_sh.at[col_b[b]],
                              ssem[b]).wait()

    def scale(b):
        def grp_body(grp, carry):
            wv = w_b[b][pl.ds(grp * 16, 16)]
            base16 = grp * 16
            for i in range(16):
                w = wv[i]
                for j in range(D // 16):
                    sl = pl.ds(j * 16, 16)
                    rows_b[b][base16 + i, sl] = rows_b[b][base16 + i, sl] * w
            return carry

        lax.fori_loop(0, C // 16, grp_body, 0)

    issue(0, 0)

    def chunk_pair(go, carry):
        for b in range(2):
            g = go * 2 + b
            nb = 1 - b

            @pl.when(g >= 1)
            def _():
                wait_scatter(nb)

            @pl.when(g + 1 < G)
            def _():
                issue(g + 1, nb)

            pltpu.make_async_copy(x_hbm.at[row_all.at[pl.ds(g * C, C)]],
                                  rows_b[b], gsem[b]).wait()
            pltpu.make_async_copy(w_hbm.at[pl.ds(ebase + g * C, C)], w_b[b],
                                  wsem[b]).wait()
            scale(b)
            pltpu.make_async_copy(col_hbm.at[pl.ds(ebase + g * C, C)],
                                  col_b[b], csem[b]).wait()
            pltpu.async_copy(rows_b[b], agg_sh.at[col_b[b]], ssem[b],
                             add=True)
        return carry

    lax.fori_loop(0, G // 2, chunk_pair, 0)
    wait_scatter(1)

    plsc.subcore_barrier()

    out_row = c * N + t0
    n_live = N - (NS - 1) * ROWS_PER_TILE

    @pl.when(s < NS - 1)
    def _():
        pltpu.sync_copy(agg_sh.at[pl.ds(t0, ROWS_PER_TILE)],
                        out_hbm.at[pl.ds(out_row, ROWS_PER_TILE)])

    @pl.when(s == NS - 1)
    def _():
        pltpu.sync_copy(agg_sh.at[pl.ds(t0, n_live)],
                        out_hbm.at[pl.ds(out_row, n_live)])


_sc_agg = pl.kernel(
    _sc_body,
    out_type=jax.ShapeDtypeStruct((NC * N, D), jnp.float32),
    mesh=plsc.VectorSubcoreMesh(core_axis_name="c", subcore_axis_name="s",
                                num_cores=NC, num_subcores=NS),
    scratch_types=[
        pltpu.VMEM_SHARED((N_SP, D), jnp.float32),
        pltpu.VMEM((EPW,), jnp.int32),
        [pltpu.VMEM((C,), jnp.float32) for _ in range(2)],
        [pltpu.VMEM((C,), jnp.int32) for _ in range(2)],
        [pltpu.VMEM((C, D), jnp.float32) for _ in range(2)],
        [pltpu.SemaphoreType.DMA for _ in range(2)],
        [pltpu.SemaphoreType.DMA for _ in range(2)],
        [pltpu.SemaphoreType.DMA for _ in range(2)],
        [pltpu.SemaphoreType.DMA for _ in range(2)],
    ],
)


BN = 1000


def _tc_body(x_ref, p0_ref, p1_ref, w1t_ref, w2t_ref, b_ref, out_ref):
    a = p0_ref[...] + p1_ref[...]
    xb = x_ref[...]
    su = a + xb
    mu = a * xb
    h = (jnp.dot(su, w1t_ref[...], preferred_element_type=jnp.float32)
         + jnp.dot(mu, w2t_ref[...], preferred_element_type=jnp.float32)
         + b_ref[...])
    out_ref[...] = jnp.where(h >= 0, h, 0.2 * h)


def _tc_dense(x, parts, w1t, w2t, b):
    nb = N // BN
    return pl.pallas_call(
        _tc_body,
        grid=(nb,),
        in_specs=[
            pl.BlockSpec((BN, D), lambda i: (i, 0)),
            pl.BlockSpec((BN, D), lambda i: (i, 0)),
            pl.BlockSpec((BN, D), lambda i: (i + nb, 0)),
            pl.BlockSpec((D, D), lambda i: (0, 0)),
            pl.BlockSpec((D, D), lambda i: (0, 0)),
            pl.BlockSpec((1, D), lambda i: (0, 0)),
        ],
        out_specs=pl.BlockSpec((BN, D), lambda i: (i, 0)),
        out_shape=jax.ShapeDtypeStruct((N, D), jnp.float32),
    )(x, parts, parts, w1t, w2t, b)


@jax.jit
def kernel(x, edge_index, edge_weight, W1, b1, W2, b2):
    row = edge_index[0].astype(jnp.int32)
    col = edge_index[1].astype(jnp.int32)
    pad = E_PAD - E
    row = jnp.concatenate([row, jnp.zeros((pad,), jnp.int32)])
    col = jnp.concatenate([col, jnp.zeros((pad,), jnp.int32)])
    w = jnp.concatenate([edge_weight, jnp.zeros((pad,), jnp.float32)])

    parts = _sc_agg(x, row, col, w)

    w1t = W1.T
    w2t = W2.T
    b = (b1 + b2).reshape(1, D)
    return _tc_dense(x, parts, w1t, w2t, b)


# speedup vs baseline: 11.6658x; 2.9194x over previous
import functools

import jax
import jax.numpy as jnp
from jax import lax
from jax.experimental import pallas as pl
from jax.experimental.pallas import tpu as pltpu
from jax.experimental.pallas import tpu_sc as plsc

N = 10000
E = 320000
D = 128

NC = 2
NS = 16
NW = NC * NS
C = 128

EPW = -(-E // (NW * 2 * C)) * 2 * C
E_PAD = EPW * NW
G = EPW // C
ROWS_PER_TILE = -(-N // (NS * 8)) * 8
N_SP = ROWS_PER_TILE * NS


def _sc_body(x_hbm, row_hbm, col_hbm, w_hbm, out_hbm,
             agg_sh, row_all, w_b, col_b, rows_b, gsem, csem, wsem, ssem):
    c = lax.axis_index("c")
    s = lax.axis_index("s")
    wid = s * NC + c
    ebase = wid * EPW

    zeros16 = jnp.zeros((16,), jnp.float32)

    def zero_row(i, carry):
        for j in range(D // 16):
            rows_b[0][i, pl.ds(j * 16, 16)] = zeros16
        return carry

    lax.fori_loop(0, C, zero_row, 0)

    t0 = s * ROWS_PER_TILE
    n_full = ROWS_PER_TILE // C
    rem = ROWS_PER_TILE - n_full * C
    for k in range(n_full):
        pltpu.sync_copy(rows_b[0], agg_sh.at[pl.ds(t0 + k * C, C)])
    if rem:
        pltpu.sync_copy(rows_b[0].at[pl.ds(0, rem)],
                        agg_sh.at[pl.ds(t0 + n_full * C, rem)])

    pltpu.sync_copy(row_hbm.at[pl.ds(ebase, EPW)], row_all)

    plsc.subcore_barrier()

    def issue(g, b):
        pltpu.async_copy(col_hbm.at[pl.ds(ebase + g * C, C)], col_b[b],
                         csem[b])
        pltpu.async_copy(w_hbm.at[pl.ds(ebase + g * C, C)], w_b[b],
                         wsem[b])
        pltpu.async_copy(x_hbm.at[row_all.at[pl.ds(g * C, C)]], rows_b[b],
                         gsem[b])

    def wait_scatter(b):
        pltpu.make_async_copy(rows_b[b], agg_sh.at[col_b[b]],
                              ssem[b]).wait()

    def scale(b):
        def grp_body(grp, carry):
            wv = w_b[b][pl.ds(grp * 16, 16)]
            base16 = grp * 16
            for i in range(16):
                w = wv[i]
                for j in range(D // 16):
                    sl = pl.ds(j * 16, 16)
                    rows_b[b][base16 + i, sl] = rows_b[b][base16 + i, sl] * w
            return carry

        lax.fori_loop(0, C // 16, grp_body, 0)

    issue(0, 0)

    def chunk_pair(go, carry):
        for b in range(2):
            g = go * 2 + b
            nb = 1 - b

            @pl.when(g >= 1)
            def _():
                wait_scatter(nb)

            @pl.when(g + 1 < G)
            def _():
                issue(g + 1, nb)

            pltpu.make_async_copy(x_hbm.at[row_all.at[pl.ds(g * C, C)]],
                                  rows_b[b], gsem[b]).wait()
            pltpu.make_async_copy(w_hbm.at[pl.ds(ebase + g * C, C)], w_b[b],
                                  wsem[b]).wait()
            scale(b)
            pltpu.make_async_copy(col_hbm.at[pl.ds(ebase + g * C, C)],
                                  col_b[b], csem[b]).wait()
            pltpu.async_copy(rows_b[b], agg_sh.at[col_b[b]], ssem[b],
                             add=True)
        return carry

    lax.fori_loop(0, G // 2, chunk_pair, 0)
    wait_scatter(1)

    plsc.subcore_barrier()

    out_row = c * N + t0
    n_live = N - (NS - 1) * ROWS_PER_TILE

    @pl.when(s < NS - 1)
    def _():
        pltpu.sync_copy(agg_sh.at[pl.ds(t0, ROWS_PER_TILE)],
                        out_hbm.at[pl.ds(out_row, ROWS_PER_TILE)])

    @pl.when(s == NS - 1)
    def _():
        pltpu.sync_copy(agg_sh.at[pl.ds(t0, n_live)],
                        out_hbm.at[pl.ds(out_row, n_live)])


_sc_agg = pl.kernel(
    _sc_body,
    out_type=jax.ShapeDtypeStruct((NC * N, D), jnp.float32),
    mesh=plsc.VectorSubcoreMesh(core_axis_name="c", subcore_axis_name="s",
                                num_cores=NC, num_subcores=NS),
    scratch_types=[
        pltpu.VMEM_SHARED((N_SP, D), jnp.float32),
        pltpu.VMEM((EPW,), jnp.int32),
        [pltpu.VMEM((C,), jnp.float32) for _ in range(2)],
        [pltpu.VMEM((C,), jnp.int32) for _ in range(2)],
        [pltpu.VMEM((C, D), jnp.float32) for _ in range(2)],
        [pltpu.SemaphoreType.DMA for _ in range(2)],
        [pltpu.SemaphoreType.DMA for _ in range(2)],
        [pltpu.SemaphoreType.DMA for _ in range(2)],
        [pltpu.SemaphoreType.DMA for _ in range(2)],
    ],
)


BN = 1000


def _tc_body(x_ref, p0_ref, p1_ref, w1t_ref, w2t_ref, b_ref, out_ref):
    a = p0_ref[...] + p1_ref[...]
    xb = x_ref[...]
    su = a + xb
    mu = a * xb
    h = (jnp.dot(su, w1t_ref[...], preferred_element_type=jnp.float32)
         + jnp.dot(mu, w2t_ref[...], preferred_element_type=jnp.float32)
         + b_ref[...])
    out_ref[...] = jnp.where(h >= 0, h, 0.2 * h)


def _tc_dense(x, parts, w1t, w2t, b):
    nb = N // BN
    return pl.pallas_call(
        _tc_body,
        grid=(nb,),
        in_specs=[
            pl.BlockSpec((BN, D), lambda i: (i, 0)),
            pl.BlockSpec((BN, D), lambda i: (i, 0)),
            pl.BlockSpec((BN, D), lambda i: (i + nb, 0)),
            pl.BlockSpec((D, D), lambda i: (0, 0)),
            pl.BlockSpec((D, D), lambda i: (0, 0)),
            pl.BlockSpec((1, D), lambda i: (0, 0)),
        ],
        out_specs=pl.BlockSpec((BN, D), lambda i: (i, 0)),
        out_shape=jax.ShapeDtypeStruct((N, D), jnp.float32),
    )(x, parts, parts, w1t, w2t, b)


@jax.jit
def kernel(x, edge_index, edge_weight, W1, b1, W2, b2):
    row = edge_index[0].astype(jnp.int32)
    col = edge_index[1].astype(jnp.int32)
    pad = E_PAD - E
    pad_idx = jnp.arange(pad, dtype=jnp.int32) % N
    row = jnp.concatenate([row, pad_idx])
    col = jnp.concatenate([col, pad_idx])
    w = jnp.concatenate([edge_weight, jnp.zeros((pad,), jnp.float32)])

    parts = _sc_agg(x, row, col, w)

    w1t = W1.T
    w2t = W2.T
    b = (b1 + b2).reshape(1, D)
    return _tc_dense(x, parts, w1t, w2t, b)


# speedup vs baseline: 12.5757x; 1.0780x over previous
import functools

import jax
import jax.numpy as jnp
from jax import lax
from jax.experimental import pallas as pl
from jax.experimental.pallas import tpu as pltpu
from jax.experimental.pallas import tpu_sc as plsc

N = 10000
E = 320000
D = 128

NC = 2
NS = 16
NW = NC * NS
C = 128

NCH = E // C
G_BASE = NCH // NW
G_REM = NCH % NW
G_MAX = G_BASE + (1 if G_REM else 0)
PAIRS = G_BASE // 2
ROWS_PER_TILE = -(-N // (NS * 8)) * 8
N_SP = ROWS_PER_TILE * NS


def _sc_body(x_hbm, ei_hbm, w_hbm, out_hbm,
             agg_sh, row_all, w_b, col_b, rows_b, gsem, csem, wsem, ssem):
    c = lax.axis_index("c")
    s = lax.axis_index("s")
    wid = s * NC + c
    n_g = G_BASE + jnp.where(wid < G_REM, 1, 0)
    ebase = (wid * G_BASE + jnp.minimum(wid, G_REM)) * C

    zeros16 = jnp.zeros((16,), jnp.float32)

    def zero_row(i, carry):
        for j in range(D // 16):
            rows_b[0][i, pl.ds(j * 16, 16)] = zeros16
        return carry

    lax.fori_loop(0, C, zero_row, 0)

    t0 = s * ROWS_PER_TILE
    n_full = ROWS_PER_TILE // C
    rem = ROWS_PER_TILE - n_full * C
    for k in range(n_full):
        pltpu.sync_copy(rows_b[0], agg_sh.at[pl.ds(t0 + k * C, C)])
    if rem:
        pltpu.sync_copy(rows_b[0].at[pl.ds(0, rem)],
                        agg_sh.at[pl.ds(t0 + n_full * C, rem)])

    @pl.when(wid < G_REM)
    def _():
        pltpu.sync_copy(ei_hbm.at[pl.ds(ebase, G_MAX * C)], row_all)

    @pl.when(wid >= G_REM)
    def _():
        pltpu.sync_copy(ei_hbm.at[pl.ds(ebase, G_BASE * C)],
                        row_all.at[pl.ds(0, G_BASE * C)])

    plsc.subcore_barrier()

    def issue(g, b):
        pltpu.async_copy(ei_hbm.at[pl.ds(E + ebase + g * C, C)], col_b[b],
                         csem[b])
        pltpu.async_copy(w_hbm.at[pl.ds(ebase + g * C, C)], w_b[b],
                         wsem[b])
        pltpu.async_copy(x_hbm.at[row_all.at[pl.ds(g * C, C)]], rows_b[b],
                         gsem[b])

    def wait_scatter(b):
        pltpu.make_async_copy(rows_b[b], agg_sh.at[col_b[b]],
                              ssem[b]).wait()

    def scale(b):
        def grp_body(grp, carry):
            wv = w_b[b][pl.ds(grp * 16, 16)]
            base16 = grp * 16
            for i in range(16):
                w = wv[i]
                for j in range(D // 16):
                    sl = pl.ds(j * 16, 16)
                    rows_b[b][base16 + i, sl] = rows_b[b][base16 + i, sl] * w
            return carry

        lax.fori_loop(0, C // 16, grp_body, 0)

    def consume(g, b):
        pltpu.make_async_copy(x_hbm.at[row_all.at[pl.ds(g * C, C)]],
                              rows_b[b], gsem[b]).wait()
        pltpu.make_async_copy(w_hbm.at[pl.ds(ebase + g * C, C)], w_b[b],
                              wsem[b]).wait()
        scale(b)
        pltpu.make_async_copy(ei_hbm.at[pl.ds(E + ebase + g * C, C)],
                              col_b[b], csem[b]).wait()
        pltpu.async_copy(rows_b[b], agg_sh.at[col_b[b]], ssem[b],
                         add=True)

    issue(0, 0)

    def chunk_pair(go, carry):
        for b in range(2):
            g = go * 2 + b
            nb = 1 - b

            @pl.when(g >= 1)
            def _():
                wait_scatter(nb)

            @pl.when(g + 1 < n_g)
            def _():
                issue(g + 1, nb)

            consume(g, b)
        return carry

    lax.fori_loop(0, PAIRS, chunk_pair, 0)

    @pl.when(wid < G_REM)
    def _():
        consume(G_BASE, 0)
        wait_scatter(0)

    wait_scatter(1)

    plsc.subcore_barrier()

    out_row = c * N + t0
    n_live = N - (NS - 1) * ROWS_PER_TILE

    @pl.when(s < NS - 1)
    def _():
        pltpu.sync_copy(agg_sh.at[pl.ds(t0, ROWS_PER_TILE)],
                        out_hbm.at[pl.ds(out_row, ROWS_PER_TILE)])

    @pl.when(s == NS - 1)
    def _():
        pltpu.sync_copy(agg_sh.at[pl.ds(t0, n_live)],
                        out_hbm.at[pl.ds(out_row, n_live)])


_sc_agg = pl.kernel(
    _sc_body,
    out_type=jax.ShapeDtypeStruct((NC * N, D), jnp.float32),
    mesh=plsc.VectorSubcoreMesh(core_axis_name="c", subcore_axis_name="s",
                                num_cores=NC, num_subcores=NS),
    scratch_types=[
        pltpu.VMEM_SHARED((N_SP, D), jnp.float32),
        pltpu.VMEM((G_MAX * C,), jnp.int32),
        [pltpu.VMEM((C,), jnp.float32) for _ in range(2)],
        [pltpu.VMEM((C,), jnp.int32) for _ in range(2)],
        [pltpu.VMEM((C, D), jnp.float32) for _ in range(2)],
        [pltpu.SemaphoreType.DMA for _ in range(2)],
        [pltpu.SemaphoreType.DMA for _ in range(2)],
        [pltpu.SemaphoreType.DMA for _ in range(2)],
        [pltpu.SemaphoreType.DMA for _ in range(2)],
    ],
)


BN = 1000


def _tc_body(x_ref, p0_ref, p1_ref, w1t_ref, w2t_ref, b_ref, out_ref):
    a = p0_ref[...] + p1_ref[...]
    xb = x_ref[...]
    su = a + xb
    mu = a * xb
    h = (jnp.dot(su, w1t_ref[...], preferred_element_type=jnp.float32)
         + jnp.dot(mu, w2t_ref[...], preferred_element_type=jnp.float32)
         + b_ref[...])
    out_ref[...] = jnp.where(h >= 0, h, 0.2 * h)


def _tc_dense(x, parts, w1t, w2t, b):
    nb = N // BN
    return pl.pallas_call(
        _tc_body,
        grid=(nb,),
        in_specs=[
            pl.BlockSpec((BN, D), lambda i: (i, 0)),
            pl.BlockSpec((BN, D), lambda i: (i, 0)),
            pl.BlockSpec((BN, D), lambda i: (i + nb, 0)),
            pl.BlockSpec((D, D), lambda i: (0, 0)),
            pl.BlockSpec((D, D), lambda i: (0, 0)),
            pl.BlockSpec((1, D), lambda i: (0, 0)),
        ],
        out_specs=pl.BlockSpec((BN, D), lambda i: (i, 0)),
        out_shape=jax.ShapeDtypeStruct((N, D), jnp.float32),
    )(x, parts, parts, w1t, w2t, b)


@jax.jit
def kernel(x, edge_index, edge_weight, W1, b1, W2, b2):
    ei_flat = edge_index.astype(jnp.int32).reshape(-1)
    parts = _sc_agg(x, ei_flat, edge_weight)

    w1t = W1.T
    w2t = W2.T
    b = (b1 + b2).reshape(1, D)
    return _tc_dense(x, parts, w1t, w2t, b)


# speedup vs baseline: 12.6808x; 1.0084x over previous
import functools

import jax
import jax.numpy as jnp
from jax import lax
from jax.experimental import pallas as pl
from jax.experimental.pallas import tpu as pltpu
from jax.experimental.pallas import tpu_sc as plsc

N = 10000
E = 320000
D = 128

NC = 2
NS = 16
NW = NC * NS
C = 128

NCH = E // C
G_BASE = NCH // NW
G_REM = NCH % NW
G_MAX = G_BASE + (1 if G_REM else 0)
PAIRS = G_BASE // 2
ROWS_PER_TILE = -(-N // (NS * 8)) * 8
N_SP = ROWS_PER_TILE * NS


def _sc_body(x_hbm, ei_hbm, w_hbm, out_hbm,
             agg_sh, row_all, w_b, col_b, rows_b, gsem, csem, wsem, ssem):
    c = lax.axis_index("c")
    s = lax.axis_index("s")
    wid = s * NC + c
    n_g = G_BASE + jnp.where(wid < G_REM, 1, 0)
    ebase = (wid * G_BASE + jnp.minimum(wid, G_REM)) * C

    zeros16 = jnp.zeros((16,), jnp.float32)

    def zero_row(i, carry):
        for j in range(D // 16):
            rows_b[0][i, pl.ds(j * 16, 16)] = zeros16
        return carry

    lax.fori_loop(0, C, zero_row, 0)

    t0 = s * ROWS_PER_TILE
    n_full = ROWS_PER_TILE // C
    rem = ROWS_PER_TILE - n_full * C
    for k in range(n_full):
        pltpu.sync_copy(rows_b[0], agg_sh.at[pl.ds(t0 + k * C, C)])
    if rem:
        pltpu.sync_copy(rows_b[0].at[pl.ds(0, rem)],
                        agg_sh.at[pl.ds(t0 + n_full * C, rem)])

    @pl.when(wid < G_REM)
    def _():
        pltpu.sync_copy(ei_hbm.at[pl.ds(ebase, G_MAX * C)], row_all)

    @pl.when(wid >= G_REM)
    def _():
        pltpu.sync_copy(ei_hbm.at[pl.ds(ebase, G_BASE * C)],
                        row_all.at[pl.ds(0, G_BASE * C)])

    plsc.subcore_barrier()

    def issue(g, b):
        pltpu.async_copy(ei_hbm.at[pl.ds(E + ebase + g * C, C)], col_b[b],
                         csem[b])
        pltpu.async_copy(w_hbm.at[pl.ds(ebase + g * C, C)], w_b[b],
                         wsem[b])
        pltpu.async_copy(x_hbm.at[row_all.at[pl.ds(g * C, C)]], rows_b[b],
                         gsem[b])

    def wait_scatter(b):
        pltpu.make_async_copy(rows_b[b], agg_sh.at[col_b[b]],
                              ssem[b]).wait()

    def scale(b):
        def grp_body(grp):
            wv = w_b[b][pl.ds(grp * 16, 16)]
            base16 = grp * 16
            for i in range(16):
                w = wv[i]
                for j in range(D // 16):
                    sl = pl.ds(j * 16, 16)
                    rows_b[b][base16 + i, sl] = rows_b[b][base16 + i, sl] * w

        plsc.parallel_loop(0, C // 16, 1, unroll=2)(grp_body)

    def consume(g, b):
        pltpu.make_async_copy(x_hbm.at[row_all.at[pl.ds(g * C, C)]],
                              rows_b[b], gsem[b]).wait()
        pltpu.make_async_copy(w_hbm.at[pl.ds(ebase + g * C, C)], w_b[b],
                              wsem[b]).wait()
        scale(b)
        pltpu.make_async_copy(ei_hbm.at[pl.ds(E + ebase + g * C, C)],
                              col_b[b], csem[b]).wait()
        pltpu.async_copy(rows_b[b], agg_sh.at[col_b[b]], ssem[b],
                         add=True)

    issue(0, 0)

    def chunk_pair(go, carry):
        for b in range(2):
            g = go * 2 + b
            nb = 1 - b

            @pl.when(g >= 1)
            def _():
                wait_scatter(nb)

            @pl.when(g + 1 < n_g)
            def _():
                issue(g + 1, nb)

            consume(g, b)
        return carry

    lax.fori_loop(0, PAIRS, chunk_pair, 0)

    @pl.when(wid < G_REM)
    def _():
        consume(G_BASE, 0)
        wait_scatter(0)

    wait_scatter(1)

    plsc.subcore_barrier()

    out_row = c * N + t0
    n_live = N - (NS - 1) * ROWS_PER_TILE

    @pl.when(s < NS - 1)
    def _():
        pltpu.sync_copy(agg_sh.at[pl.ds(t0, ROWS_PER_TILE)],
                        out_hbm.at[pl.ds(out_row, ROWS_PER_TILE)])

    @pl.when(s == NS - 1)
    def _():
        pltpu.sync_copy(agg_sh.at[pl.ds(t0, n_live)],
                        out_hbm.at[pl.ds(out_row, n_live)])


_sc_agg = pl.kernel(
    _sc_body,
    out_type=jax.ShapeDtypeStruct((NC * N, D), jnp.float32),
    mesh=plsc.VectorSubcoreMesh(core_axis_name="c", subcore_axis_name="s",
                                num_cores=NC, num_subcores=NS),
    scratch_types=[
        pltpu.VMEM_SHARED((N_SP, D), jnp.float32),
        pltpu.VMEM((G_MAX * C,), jnp.int32),
        [pltpu.VMEM((C,), jnp.float32) for _ in range(2)],
        [pltpu.VMEM((C,), jnp.int32) for _ in range(2)],
        [pltpu.VMEM((C, D), jnp.float32) for _ in range(2)],
        [pltpu.SemaphoreType.DMA for _ in range(2)],
        [pltpu.SemaphoreType.DMA for _ in range(2)],
        [pltpu.SemaphoreType.DMA for _ in range(2)],
        [pltpu.SemaphoreType.DMA for _ in range(2)],
    ],
)


BN = 1000


def _tc_body(x_ref, p0_ref, p1_ref, w1t_ref, w2t_ref, b_ref, out_ref):
    a = p0_ref[...] + p1_ref[...]
    xb = x_ref[...]
    su = a + xb
    mu = a * xb
    h = (jnp.dot(su, w1t_ref[...], preferred_element_type=jnp.float32)
         + jnp.dot(mu, w2t_ref[...], preferred_element_type=jnp.float32)
         + b_ref[...])
    out_ref[...] = jnp.where(h >= 0, h, 0.2 * h)


def _tc_dense(x, parts, w1t, w2t, b):
    nb = N // BN
    return pl.pallas_call(
        _tc_body,
        grid=(nb,),
        in_specs=[
            pl.BlockSpec((BN, D), lambda i: (i, 0)),
            pl.BlockSpec((BN, D), lambda i: (i, 0)),
            pl.BlockSpec((BN, D), lambda i: (i + nb, 0)),
            pl.BlockSpec((D, D), lambda i: (0, 0)),
            pl.BlockSpec((D, D), lambda i: (0, 0)),
            pl.BlockSpec((1, D), lambda i: (0, 0)),
        ],
        out_specs=pl.BlockSpec((BN, D), lambda i: (i, 0)),
        out_shape=jax.ShapeDtypeStruct((N, D), jnp.float32),
    )(x, parts, parts, w1t, w2t, b)


@jax.jit
def kernel(x, edge_index, edge_weight, W1, b1, W2, b2):
    ei_flat = edge_index.astype(jnp.int32).reshape(-1)
    parts = _sc_agg(x, ei_flat, edge_weight)

    w1t = W1.T
    w2t = W2.T
    b = (b1 + b2).reshape(1, D)
    return _tc_dense(x, parts, w1t, w2t, b)


# speedup vs baseline: 12.8668x; 1.0147x over previous
import jax
import jax.numpy as jnp
from jax import lax
from jax.experimental import pallas as pl
from jax.experimental.pallas import tpu as pltpu
from jax.experimental.pallas import tpu_sc as plsc

N = 10000
E = 320000
D = 128

NC = 2
NS = 16
NW = NC * NS
C = 128

NCH = E // C
G_BASE = NCH // NW
G_REM = NCH % NW
G_MAX = G_BASE + (1 if G_REM else 0)
PAIRS = G_BASE // 2

ROWS_PER_TILE = -(-N // (NS * 8)) * 8
N_SP = ROWS_PER_TILE * NS


def _sc_body(x_hbm, ei_hbm, w_hbm, out_hbm,
             agg_sh, row_all, w_b, col_b, rows_b, gsem, csem, wsem, ssem):
    c = lax.axis_index("c")
    s = lax.axis_index("s")
    wid = s * NC + c
    n_g = G_BASE + jnp.where(wid < G_REM, 1, 0)
    ebase = (wid * G_BASE + jnp.minimum(wid, G_REM)) * C

    zeros16 = jnp.zeros((16,), jnp.float32)

    def zero_row(i, carry):
        for j in range(D // 16):
            rows_b[0][i, pl.ds(j * 16, 16)] = zeros16
        return carry

    lax.fori_loop(0, C, zero_row, 0)

    t0 = s * ROWS_PER_TILE
    n_full = ROWS_PER_TILE // C
    rem = ROWS_PER_TILE - n_full * C
    for k in range(n_full):
        pltpu.sync_copy(rows_b[0], agg_sh.at[pl.ds(t0 + k * C, C)])
    if rem:
        pltpu.sync_copy(rows_b[0].at[pl.ds(0, rem)],
                        agg_sh.at[pl.ds(t0 + n_full * C, rem)])

    @pl.when(wid < G_REM)
    def _():
        pltpu.sync_copy(ei_hbm.at[pl.ds(ebase, G_MAX * C)], row_all)

    @pl.when(wid >= G_REM)
    def _():
        pltpu.sync_copy(ei_hbm.at[pl.ds(ebase, G_BASE * C)],
                        row_all.at[pl.ds(0, G_BASE * C)])

    plsc.subcore_barrier()

    def issue(g, b):
        pltpu.async_copy(ei_hbm.at[pl.ds(E + ebase + g * C, C)], col_b[b],
                         csem[b])
        pltpu.async_copy(w_hbm.at[pl.ds(ebase + g * C, C)], w_b[b],
                         wsem[b])
        pltpu.async_copy(x_hbm.at[row_all.at[pl.ds(g * C, C)]], rows_b[b],
                         gsem[b])

    def wait_scatter(b):
        pltpu.make_async_copy(rows_b[b], agg_sh.at[col_b[b]],
                              ssem[b]).wait()

    def scale(b):
        def grp_body(grp):
            wv = w_b[b][pl.ds(grp * 16, 16)]
            base16 = grp * 16
            for i in range(16):
                w = wv[i]
                for j in range(D // 16):
                    sl = pl.ds(j * 16, 16)
                    rows_b[b][base16 + i, sl] = rows_b[b][base16 + i, sl] * w

        plsc.parallel_loop(0, C // 16, 1, unroll=2)(grp_body)

    def consume(g, b):
        pltpu.make_async_copy(x_hbm.at[row_all.at[pl.ds(g * C, C)]],
                              rows_b[b], gsem[b]).wait()
        pltpu.make_async_copy(w_hbm.at[pl.ds(ebase + g * C, C)], w_b[b],
                              wsem[b]).wait()
        scale(b)
        pltpu.make_async_copy(ei_hbm.at[pl.ds(E + ebase + g * C, C)],
                              col_b[b], csem[b]).wait()
        pltpu.async_copy(rows_b[b], agg_sh.at[col_b[b]], ssem[b],
                         add=True)

    issue(0, 0)

    def chunk_pair(go, carry):
        for b in range(2):
            g = go * 2 + b
            nb = 1 - b

            @pl.when(g >= 1)
            def _():
                wait_scatter(nb)

            @pl.when(g + 1 < n_g)
            def _():
                issue(g + 1, nb)

            consume(g, b)
        return carry

    lax.fori_loop(0, PAIRS, chunk_pair, 0)

    @pl.when(wid < G_REM)
    def _():
        consume(G_BASE, 0)
        wait_scatter(0)

    wait_scatter(1)

    plsc.subcore_barrier()

    out_row = c * N + t0
    n_live = N - (NS - 1) * ROWS_PER_TILE

    @pl.when(s < NS - 1)
    def _():
        pltpu.sync_copy(agg_sh.at[pl.ds(t0, ROWS_PER_TILE)],
                        out_hbm.at[pl.ds(out_row, ROWS_PER_TILE)])

    @pl.when(s == NS - 1)
    def _():
        pltpu.sync_copy(agg_sh.at[pl.ds(t0, n_live)],
                        out_hbm.at[pl.ds(out_row, n_live)])


_sc_agg = pl.kernel(
    _sc_body,
    out_type=jax.ShapeDtypeStruct((NC * N, D), jnp.float32),
    mesh=plsc.VectorSubcoreMesh(core_axis_name="c", subcore_axis_name="s",
                                num_cores=NC, num_subcores=NS),
    scratch_types=[
        pltpu.VMEM_SHARED((N_SP, D), jnp.float32),
        pltpu.VMEM((G_MAX * C,), jnp.int32),
        [pltpu.VMEM((C,), jnp.float32) for _ in range(2)],
        [pltpu.VMEM((C,), jnp.int32) for _ in range(2)],
        [pltpu.VMEM((C, D), jnp.float32) for _ in range(2)],
        [pltpu.SemaphoreType.DMA for _ in range(2)],
        [pltpu.SemaphoreType.DMA for _ in range(2)],
        [pltpu.SemaphoreType.DMA for _ in range(2)],
        [pltpu.SemaphoreType.DMA for _ in range(2)],
    ],
)


BN = 2000


def _tc_body(x_ref, p0_ref, p1_ref, w1t_ref, w2t_ref, b_ref, out_ref):
    a = p0_ref[...] + p1_ref[...]
    xb = x_ref[...]
    su = a + xb
    mu = a * xb
    h = (jnp.dot(su, w1t_ref[...], preferred_element_type=jnp.float32)
         + jnp.dot(mu, w2t_ref[...], preferred_element_type=jnp.float32)
         + b_ref[...])
    out_ref[...] = jnp.where(h >= 0, h, 0.2 * h)


def _tc_dense(x, parts, w1t, w2t, b):
    nb = N // BN
    return pl.pallas_call(
        _tc_body,
        grid=(nb,),
        in_specs=[
            pl.BlockSpec((BN, D), lambda i: (i, 0)),
            pl.BlockSpec((BN, D), lambda i: (i, 0)),
            pl.BlockSpec((BN, D), lambda i: (i + nb, 0)),
            pl.BlockSpec((D, D), lambda i: (0, 0)),
            pl.BlockSpec((D, D), lambda i: (0, 0)),
            pl.BlockSpec((1, D), lambda i: (0, 0)),
        ],
        out_specs=pl.BlockSpec((BN, D), lambda i: (i, 0)),
        out_shape=jax.ShapeDtypeStruct((N, D), jnp.float32),
    )(x, parts, parts, w1t, w2t, b)


@jax.jit
def kernel(x, edge_index, edge_weight, W1, b1, W2, b2):
    ei_flat = edge_index.astype(jnp.int32).reshape(-1)
    parts = _sc_agg(x, ei_flat, edge_weight)

    w1t = W1.T
    w2t = W2.T
    b = (b1 + b2).reshape(1, D)
    return _tc_dense(x, parts, w1t, w2t, b)


# speedup vs baseline: 13.3788x; 1.0398x over previous
import jax
import jax.numpy as jnp
from jax import lax
from jax.experimental import pallas as pl
from jax.experimental.pallas import tpu as pltpu
from jax.experimental.pallas import tpu_sc as plsc

N = 10000
E = 320000
D = 128

NC = 2
NS = 16
NW = NC * NS
C = 128
NB = 3

NCH = E // C
G_BASE = NCH // NW
G_REM = NCH % NW
TRIPLES = G_BASE // NB

RPT = (N // NS) // 8 * 8
RPT_LAST = N - (NS - 1) * RPT


def _sc_body(x_hbm, ei_hbm, w_hbm, out_hbm,
             agg_sh, row_b, w_b, col_b, rows_b, rsem, gsem, csem, wsem,
             ssem):
    c = lax.axis_index("c")
    s = lax.axis_index("s")
    wid = s * NC + c
    n_g = G_BASE + jnp.where(wid < G_REM, 1, 0)
    ebase = (wid * G_BASE + jnp.minimum(wid, G_REM)) * C

    zeros16 = jnp.zeros((16,), jnp.float32)

    def zero_row(i, carry):
        for j in range(D // 16):
            rows_b[0][i, pl.ds(j * 16, 16)] = zeros16
        return carry

    lax.fori_loop(0, C, zero_row, 0)

    t0 = s * RPT

    @pl.when(s < NS - 1)
    def _():
        for k in range(RPT // C):
            pltpu.sync_copy(rows_b[0], agg_sh.at[pl.ds(t0 + k * C, C)])
        rem = RPT - (RPT // C) * C
        if rem:
            pltpu.sync_copy(rows_b[0].at[pl.ds(0, rem)],
                            agg_sh.at[pl.ds(t0 + (RPT // C) * C, rem)])

    @pl.when(s == NS - 1)
    def _():
        for k in range(RPT_LAST // C):
            pltpu.sync_copy(rows_b[0], agg_sh.at[pl.ds(t0 + k * C, C)])
        rem = RPT_LAST - (RPT_LAST // C) * C
        if rem:
            pltpu.sync_copy(rows_b[0].at[pl.ds(0, rem)],
                            agg_sh.at[pl.ds(t0 + (RPT_LAST // C) * C, rem)])

    plsc.subcore_barrier()

    def issue_idx(g, b):
        pltpu.async_copy(ei_hbm.at[pl.ds(ebase + g * C, C)], row_b[b],
                         rsem[b])
        pltpu.async_copy(ei_hbm.at[pl.ds(E + ebase + g * C, C)], col_b[b],
                         csem[b])
        pltpu.async_copy(w_hbm.at[pl.ds(ebase + g * C, C)], w_b[b],
                         wsem[b])

    def issue_gather(g, b):
        pltpu.make_async_copy(ei_hbm.at[pl.ds(ebase + g * C, C)], row_b[b],
                              rsem[b]).wait()
        pltpu.async_copy(x_hbm.at[row_b[b]], rows_b[b], gsem[b])

    def wait_scatter(b):
        pltpu.make_async_copy(rows_b[b], agg_sh.at[col_b[b]],
                              ssem[b]).wait()

    def scale(b):
        def grp_body(grp):
            wv = w_b[b][pl.ds(grp * 16, 16)]
            base16 = grp * 16
            for i in range(16):
                w = wv[i]
                for j in range(D // 16):
                    sl = pl.ds(j * 16, 16)
                    rows_b[b][base16 + i, sl] = rows_b[b][base16 + i, sl] * w

        plsc.parallel_loop(0, C // 16, 1, unroll=2)(grp_body)

    def consume(g, b):
        pltpu.make_async_copy(x_hbm.at[row_b[b]], rows_b[b],
                              gsem[b]).wait()
        pltpu.make_async_copy(w_hbm.at[pl.ds(ebase + g * C, C)], w_b[b],
                              wsem[b]).wait()
        scale(b)
        pltpu.make_async_copy(ei_hbm.at[pl.ds(E + ebase + g * C, C)],
                              col_b[b], csem[b]).wait()
        pltpu.async_copy(rows_b[b], agg_sh.at[col_b[b]], ssem[b],
                         add=True)

    issue_idx(0, 0)
    issue_gather(0, 0)
    issue_idx(1, 1)
    issue_gather(1, 1)

    def chunk_triple(go, carry):
        for b in range(NB):
            g = go * NB + b
            nb = (b + 2) % NB

            @pl.when(g >= 1)
            def _():
                wait_scatter(nb)

            @pl.when(g + 2 < n_g)
            def _():
                issue_idx(g + 2, nb)

            consume(g, b)

            @pl.when(g + 2 < n_g)
            def _():
                issue_gather(g + 2, nb)
        return carry

    lax.fori_loop(0, TRIPLES, chunk_triple, 0)

    @pl.when(wid < G_REM)
    def _():
        consume(G_BASE, 0)
        wait_scatter(0)

    wait_scatter(2)

    plsc.subcore_barrier()

    out_row = c * N + t0

    @pl.when(s < NS - 1)
    def _():
        pltpu.sync_copy(agg_sh.at[pl.ds(t0, RPT)],
                        out_hbm.at[pl.ds(out_row, RPT)])

    @pl.when(s == NS - 1)
    def _():
        pltpu.sync_copy(agg_sh.at[pl.ds(t0, RPT_LAST)],
                        out_hbm.at[pl.ds(out_row, RPT_LAST)])


_sc_agg = pl.kernel(
    _sc_body,
    out_type=jax.ShapeDtypeStruct((NC * N, D), jnp.float32),
    mesh=plsc.VectorSubcoreMesh(core_axis_name="c", subcore_axis_name="s",
                                num_cores=NC, num_subcores=NS),
    scratch_types=[
        pltpu.VMEM_SHARED((N, D), jnp.float32),
        [pltpu.VMEM((C,), jnp.int32) for _ in range(NB)],
        [pltpu.VMEM((C,), jnp.float32) for _ in range(NB)],
        [pltpu.VMEM((C,), jnp.int32) for _ in range(NB)],
        [pltpu.VMEM((C, D), jnp.float32) for _ in range(NB)],
        [pltpu.SemaphoreType.DMA for _ in range(NB)],
        [pltpu.SemaphoreType.DMA for _ in range(NB)],
        [pltpu.SemaphoreType.DMA for _ in range(NB)],
        [pltpu.SemaphoreType.DMA for _ in range(NB)],
        [pltpu.SemaphoreType.DMA for _ in range(NB)],
    ],
)


BN = 2000


def _tc_body(x_ref, p0_ref, p1_ref, w1t_ref, w2t_ref, b_ref, out_ref):
    a = p0_ref[...] + p1_ref[...]
    xb = x_ref[...]
    su = a + xb
    mu = a * xb
    h = (jnp.dot(su, w1t_ref[...], preferred_element_type=jnp.float32)
         + jnp.dot(mu, w2t_ref[...], preferred_element_type=jnp.float32)
         + b_ref[...])
    out_ref[...] = jnp.where(h >= 0, h, 0.2 * h)


def _tc_dense(x, parts, w1t, w2t, b):
    nb = N // BN
    return pl.pallas_call(
        _tc_body,
        grid=(nb,),
        in_specs=[
            pl.BlockSpec((BN, D), lambda i: (i, 0)),
            pl.BlockSpec((BN, D), lambda i: (i, 0)),
            pl.BlockSpec((BN, D), lambda i: (i + nb, 0)),
            pl.BlockSpec((D, D), lambda i: (0, 0)),
            pl.BlockSpec((D, D), lambda i: (0, 0)),
            pl.BlockSpec((1, D), lambda i: (0, 0)),
        ],
        out_specs=pl.BlockSpec((BN, D), lambda i: (i, 0)),
        out_shape=jax.ShapeDtypeStruct((N, D), jnp.float32),
    )(x, parts, parts, w1t, w2t, b)


@jax.jit
def kernel(x, edge_index, edge_weight, W1, b1, W2, b2):
    ei_flat = edge_index.astype(jnp.int32).reshape(-1)
    parts = _sc_agg(x, ei_flat, edge_weight)

    w1t = W1.T
    w2t = W2.T
    b = (b1 + b2).reshape(1, D)
    return _tc_dense(x, parts, w1t, w2t, b)
